# baseline (speedup 1.0000x reference)
import functools
import jax
import jax.numpy as jnp
from jax import lax
from jax.experimental import pallas as pl
from jax.experimental.pallas import tpu as pltpu
from jax.experimental.pallas import tpu_sc as plsc

N_E = 320000
N_N = 10000
N_NP = 10240
_EB = 3200
_NBLK = N_E // _EB
_NB = 128
_NROW = N_E // _NB
_NROWP = 2560
_GPT = 20
_SLAB = N_NP // 16
_PAD_IDX = 10200


def _k1_body(ea, w1, b1, w2, b2, e1_o, e2_o, se1_o, se2_o):
    i = pl.program_id(0)
    e1 = jax.nn.relu(jnp.dot(ea[...], w1[...], preferred_element_type=jnp.float32) + b1[...])
    e2 = jax.nn.relu(jnp.dot(e1, w2[...], preferred_element_type=jnp.float32) + b2[...])
    e1_o[...] = e1
    e2_o[...] = e2
    s1 = jnp.sum(e1, axis=0, keepdims=True)
    s2 = jnp.sum(e2, axis=0, keepdims=True)

    @pl.when(i == 0)
    def _():
        se1_o[...] = s1
        se2_o[...] = s2

    @pl.when(i > 0)
    def _():
        se1_o[...] += s1
        se2_o[...] += s2


def _k1(ea, w1, b1, w2, b2):
    return pl.pallas_call(
        _k1_body,
        grid=(_NBLK,),
        in_specs=[
            pl.BlockSpec((_EB, 16), lambda i: (i, 0)),
            pl.BlockSpec((16, 256), lambda i: (0, 0)),
            pl.BlockSpec((1, 256), lambda i: (0, 0)),
            pl.BlockSpec((256, 128), lambda i: (0, 0)),
            pl.BlockSpec((1, 128), lambda i: (0, 0)),
        ],
        out_specs=[
            pl.BlockSpec((_EB, 256), lambda i: (i, 0)),
            pl.BlockSpec((_EB, 128), lambda i: (i, 0)),
            pl.BlockSpec((1, 256), lambda i: (0, 0)),
            pl.BlockSpec((1, 128), lambda i: (0, 0)),
        ],
        out_shape=[
            jax.ShapeDtypeStruct((N_E, 256), jnp.float32),
            jax.ShapeDtypeStruct((N_E, 128), jnp.float32),
            jax.ShapeDtypeStruct((1, 256), jnp.float32),
            jax.ShapeDtypeStruct((1, 128), jnp.float32),
        ],
        compiler_params=pltpu.CompilerParams(
            dimension_semantics=("arbitrary",)),
    )(ea, w1, b1, w2, b2)


def _sc_body(e1, e2, dst2d, src2d, zer,
             o_inc1, o_out1, o_inc2, o_out2, o_dinc, o_dout,
             rows_a, rows_b, idx_v, acc, sem_a, sem_b, sem_sa, sem_sb):
    c = lax.axis_index("c")
    s = lax.axis_index("s")
    r0 = s * _SLAB
    b0 = s * _GPT * 8
    bL = b0 + _GPT * 8 - 1
    one16 = jnp.ones((16,), jnp.float32)

    def _pass(mat_ref, col0, idx2d, out_ref, ocol0):
        pltpu.sync_copy(zer, acc.at[pl.ds(r0, _SLAB)])
        plsc.subcore_barrier()

        def _rows_src(r):
            base = jnp.minimum(r, _NROW - 1) * _NB
            return mat_ref.at[pl.ds(base, _NB), pl.ds(col0, 128)]

        def _wait_scat(buf, ssem, jj):
            pltpu.make_async_copy(buf, acc.at[idx_v.at[jj]], ssem).wait()

        pltpu.async_copy(_rows_src(b0), rows_a, sem_a)

        def _dgroup(g2, carry):
            grow0 = (s * _GPT + 2 * g2) * 8

            @pl.when(g2 > 0)
            def _():
                _wait_scat(rows_a, sem_sa, 0)
                _wait_scat(rows_b, sem_sb, 1)

            pltpu.sync_copy(idx2d.at[pl.ds(grow0, 16)], idx_v)
            for jj in range(16):
                r = grow0 + jj
                cur, csem, cssem = ((rows_a, sem_a, sem_sa) if jj % 2 == 0
                                    else (rows_b, sem_b, sem_sb))
                nxt, nsem, nssem = ((rows_b, sem_b, sem_sb) if jj % 2 == 0
                                    else (rows_a, sem_a, sem_sa))
                if jj >= 2:
                    _wait_scat(nxt, nssem, jj - 1)
                if jj < 15:
                    pltpu.async_copy(_rows_src(r + 1), nxt, nsem)
                else:
                    @pl.when(r < bL)
                    def _():
                        pltpu.async_copy(_rows_src(r + 1), nxt, nsem)
                pltpu.make_async_copy(_rows_src(r), cur, csem).wait()
                pltpu.async_copy(cur, acc.at[idx_v.at[jj]], cssem)
            return carry

        lax.fori_loop(0, _GPT // 2, _dgroup, 0)
        _wait_scat(rows_a, sem_sa, 14)
        _wait_scat(rows_b, sem_sb, 15)
        plsc.subcore_barrier()
        pltpu.sync_copy(acc.at[pl.ds(r0, _SLAB)],
                        out_ref.at[pl.ds(r0, _SLAB), pl.ds(ocol0, 128)])

    def _deg_pass(idx2d, out_ref):
        pltpu.sync_copy(zer, acc.at[pl.ds(r0, _SLAB)])
        plsc.subcore_barrier()

        def _group(g2, carry):
            grow0 = (s * _GPT + 2 * g2) * 8
            pltpu.sync_copy(idx2d.at[pl.ds(grow0, 16)], idx_v)
            for jj in range(16):
                pltpu.sync_copy(rows_a, acc.at[idx_v.at[jj]], add=True)
            return carry

        lax.fori_loop(0, _GPT // 2, _group, 0)
        plsc.subcore_barrier()
        pltpu.sync_copy(acc.at[pl.ds(r0, _SLAB)], out_ref.at[pl.ds(r0, _SLAB)])

    def _fill_ones(r, carry):
        for k2 in range(8):
            rows_a[r, pl.ds(16 * k2, 16)] = one16
        return carry

    lax.fori_loop(0, _NB, _fill_ones, 0)

    @pl.when(c == 0)
    def _():
        _deg_pass(dst2d, o_dinc)
        _pass(e1, 0, dst2d, o_inc1, 0)
        _pass(e1, 0, src2d, o_out1, 0)
        _pass(e1, 128, dst2d, o_inc1, 128)

    @pl.when(c == 1)
    def _():
        _deg_pass(src2d, o_dout)
        _pass(e1, 128, src2d, o_out1, 128)
        _pass(e2, 0, dst2d, o_inc2, 0)
        _pass(e2, 0, src2d, o_out2, 0)


@functools.lru_cache(maxsize=None)
def _make_sc_scatter():
    return pl.kernel(
        _sc_kernel_fn,
        mesh=plsc.VectorSubcoreMesh(core_axis_name="c", subcore_axis_name="s"),
        out_type=[
            jax.ShapeDtypeStruct((N_NP, 256), jnp.float32),
            jax.ShapeDtypeStruct((N_NP, 256), jnp.float32),
            jax.ShapeDtypeStruct((N_NP, 128), jnp.float32),
            jax.ShapeDtypeStruct((N_NP, 128), jnp.float32),
            jax.ShapeDtypeStruct((N_NP, 128), jnp.float32),
            jax.ShapeDtypeStruct((N_NP, 128), jnp.float32),
        ],
        scratch_types=[
            pltpu.VMEM((_NB, 128), jnp.float32),
            pltpu.VMEM((_NB, 128), jnp.float32),
            pltpu.VMEM((16, _NB), jnp.int32),
            pltpu.VMEM_SHARED((N_NP, 128), jnp.float32),
            pltpu.SemaphoreType.DMA,
            pltpu.SemaphoreType.DMA,
            pltpu.SemaphoreType.DMA,
            pltpu.SemaphoreType.DMA,
        ],
    )


def _sc_kernel_fn(e1, e2, dst2d, src2d, zer,
                  o_inc1, o_out1, o_inc2, o_out2, o_dinc, o_dout,
                  rows_a, rows_b, idx_v, acc, sem_a, sem_b, sem_sa, sem_sb):
    _sc_body(e1, e2, dst2d, src2d, zer,
             o_inc1, o_out1, o_inc2, o_out2, o_dinc, o_dout,
             rows_a, rows_b, idx_v, acc, sem_a, sem_b, sem_sa, sem_sb)


_NNB = 2000
_NNBLK = N_N // _NNB


def _k3_body(x, inc1, out1, inc2, out2, dinc, dout,
             wn1, win1, wout1, bn1, wn2, win2, wout2, bn2,
             se1, se2, u, a,
             wu1, wun1, wue1, bu1, wu2, wun2, wue2, bu2,
             wga, bga, wa1, ba1, wa2, ba2, wa3, ba3,
             q_o, sn1_v, sn2_v):
    i = pl.program_id(0)
    dot = lambda p, w: jnp.dot(p, w, preferred_element_type=jnp.float32)
    inv_di = 1.0 / jnp.maximum(dinc[...][:, 0:1], 1.0)
    inv_do = 1.0 / jnp.maximum(dout[...][:, 0:1], 1.0)

    n1 = dot(x[...], wn1[...])
    n1 += dot(inc1[...] * inv_di, win1[...])
    n1 += dot(out1[...] * inv_do, wout1[...])
    n1 = jax.nn.relu(n1 + bn1[...])

    n2 = dot(n1, wn2[...])
    n2 += dot(inc2[...] * inv_di, win2[...])
    n2 += dot(out2[...] * inv_do, wout2[...])
    n2 = jax.nn.relu(n2 + bn2[...])

    s1 = jnp.sum(n1, axis=0, keepdims=True)
    s2 = jnp.sum(n2, axis=0, keepdims=True)

    @pl.when(i == 0)
    def _():
        sn1_v[...] = s1
        sn2_v[...] = s2

    @pl.when(i > 0)
    def _():
        sn1_v[...] += s1
        sn2_v[...] += s2

    @pl.when(i == _NNBLK - 1)
    def _():
        mn1 = sn1_v[...] * (1.0 / N_N)
        mn2 = sn2_v[...] * (1.0 / N_N)
        me1 = se1[...] * (1.0 / N_E)
        me2 = se2[...] * (1.0 / N_E)
        u1 = jax.nn.relu(dot(u[...], wu1[...]) + dot(mn1, wun1[...])
                         + dot(me1, wue1[...]) + bu1[...])
        u2 = jax.nn.relu(dot(u1, wu2[...]) + dot(mn2, wun2[...])
                         + dot(me2, wue2[...]) + bu2[...])
        sv = dot(u2, wga[...]) + bga[...]
        h = jax.nn.relu(dot(sv, wa1[0:128, :]) + dot(a[...], wa1[128:160, :]) + ba1[...])
        h = jax.nn.relu(dot(h, wa2[...]) + ba2[...])
        q_o[...] = dot(h, wa3[...]) + ba3[...]


def _k3(x, inc1, out1, inc2, out2, dinc, dout, *weights):
    def whole(shape):
        return pl.BlockSpec(shape, lambda i, _n=len(shape): (0,) * _n)

    in_specs = [
        pl.BlockSpec((_NNB, 128), lambda i: (i, 0)),
        pl.BlockSpec((_NNB, 256), lambda i: (i, 0)),
        pl.BlockSpec((_NNB, 256), lambda i: (i, 0)),
        pl.BlockSpec((_NNB, 128), lambda i: (i, 0)),
        pl.BlockSpec((_NNB, 128), lambda i: (i, 0)),
        pl.BlockSpec((_NNB, 128), lambda i: (i, 0)),
        pl.BlockSpec((_NNB, 128), lambda i: (i, 0)),
    ] + [whole(w.shape) for w in weights]
    return pl.pallas_call(
        _k3_body,
        grid=(_NNBLK,),
        in_specs=in_specs,
        out_specs=pl.BlockSpec((1, 1), lambda i: (0, 0)),
        out_shape=jax.ShapeDtypeStruct((1, 1), jnp.float32),
        scratch_shapes=[pltpu.VMEM((1, 256), jnp.float32),
                        pltpu.VMEM((1, 128), jnp.float32)],
        compiler_params=pltpu.CompilerParams(
            dimension_semantics=("arbitrary",)),
    )(x, inc1, out1, inc2, out2, dinc, dout, *weights)


def kernel(x, edge_index, edge_attr, u, a,
           W_e1, b_e1,
           W_n1, W_in1, W_out1, b_n1,
           W_u1, W_un1, W_ue1, b_u1,
           W_e2, b_e2,
           W_n2, W_in2, W_out2, b_n2,
           W_u2, W_un2, W_ue2, b_u2,
           W_ga, b_ga,
           W_a1, b_a1, W_a2, b_a2, W_a3, b_a3):
    e1, e2, se1, se2 = _k1(edge_attr, W_e1, b_e1.reshape(1, -1),
                           W_e2, b_e2.reshape(1, -1))
    pad = jnp.full((_NROWP * _NB - N_E,), _PAD_IDX, jnp.int32)
    dst2d = jnp.concatenate([edge_index[1], pad]).reshape(_NROWP, _NB)
    src2d = jnp.concatenate([edge_index[0], pad]).reshape(_NROWP, _NB)
    zer = jnp.zeros((_SLAB, 128), jnp.float32)
    inc1, out1, inc2, out2, dinc, dout = _make_sc_scatter()(e1, e2, dst2d, src2d, zer)
    return _k3(x, inc1, out1, inc2, out2, dinc, dout,
               W_n1, W_in1, W_out1, b_n1.reshape(1, -1),
               W_n2, W_in2, W_out2, b_n2.reshape(1, -1),
               se1, se2, u, a,
               W_u1, W_un1, W_ue1, b_u1.reshape(1, -1),
               W_u2, W_un2, W_ue2, b_u2.reshape(1, -1),
               W_ga, b_ga.reshape(1, -1),
               W_a1, b_a1.reshape(1, -1), W_a2, b_a2.reshape(1, -1),
               W_a3, b_a3.reshape(1, -1))


# speedup vs baseline: 1.0933x; 1.0933x over previous
import functools
import jax
import jax.numpy as jnp
from jax import lax
from jax.experimental import pallas as pl
from jax.experimental.pallas import tpu as pltpu
from jax.experimental.pallas import tpu_sc as plsc

N_E = 320000
N_N = 10000
N_NP = 10240
_EB = 3200
_NBLK = N_E // _EB
_NB = 128
_NROW = N_E // _NB
_NROWP = 2560
_GPT = 20
_SLAB = N_NP // 16
_PAD_IDX = 10200


def _k1_body(ea, w1, b1, w2, b2, e1_o, e2_o, se1_o, se2_o):
    i = pl.program_id(0)
    e1 = jax.nn.relu(jnp.dot(ea[...], w1[...], preferred_element_type=jnp.float32) + b1[...])
    e2 = jax.nn.relu(jnp.dot(e1, w2[...], preferred_element_type=jnp.float32) + b2[...])
    e1_o[...] = e1
    e2_o[...] = e2
    s1 = jnp.sum(e1, axis=0, keepdims=True)
    s2 = jnp.sum(e2, axis=0, keepdims=True)

    @pl.when(i == 0)
    def _():
        se1_o[...] = s1
        se2_o[...] = s2

    @pl.when(i > 0)
    def _():
        se1_o[...] += s1
        se2_o[...] += s2


def _k1(ea, w1, b1, w2, b2):
    return pl.pallas_call(
        _k1_body,
        grid=(_NBLK,),
        in_specs=[
            pl.BlockSpec((_EB, 16), lambda i: (i, 0)),
            pl.BlockSpec((16, 256), lambda i: (0, 0)),
            pl.BlockSpec((1, 256), lambda i: (0, 0)),
            pl.BlockSpec((256, 128), lambda i: (0, 0)),
            pl.BlockSpec((1, 128), lambda i: (0, 0)),
        ],
        out_specs=[
            pl.BlockSpec((_EB, 256), lambda i: (i, 0)),
            pl.BlockSpec((_EB, 128), lambda i: (i, 0)),
            pl.BlockSpec((1, 256), lambda i: (0, 0)),
            pl.BlockSpec((1, 128), lambda i: (0, 0)),
        ],
        out_shape=[
            jax.ShapeDtypeStruct((N_E, 256), jnp.float32),
            jax.ShapeDtypeStruct((N_E, 128), jnp.float32),
            jax.ShapeDtypeStruct((1, 256), jnp.float32),
            jax.ShapeDtypeStruct((1, 128), jnp.float32),
        ],
        compiler_params=pltpu.CompilerParams(
            dimension_semantics=("arbitrary",)),
    )(ea, w1, b1, w2, b2)


def _sc_body(e1, e2, dst2d, src2d, zer,
             o_inc1, o_out1, o_inc2, o_out2, o_dinc, o_dout,
             rows_a, rows_b, idx_v, acc, sem_a, sem_b):
    c = lax.axis_index("c")
    s = lax.axis_index("s")
    r0 = s * _SLAB
    b0 = s * _GPT * 8
    bL = b0 + _GPT * 8 - 1
    one16 = jnp.ones((16,), jnp.float32)

    def _pass(mat_ref, col0, idx2d, out_ref, ocol0):
        pltpu.sync_copy(zer, acc.at[pl.ds(r0, _SLAB)])
        plsc.subcore_barrier()

        def _rows_src(r):
            base = jnp.minimum(r, _NROW - 1) * _NB
            return mat_ref.at[pl.ds(base, _NB), pl.ds(col0, 128)]

        pltpu.async_copy(_rows_src(b0), rows_a, sem_a)

        def _dgroup(g2, carry):
            grow0 = (s * _GPT + 2 * g2) * 8
            pltpu.sync_copy(idx2d.at[pl.ds(grow0, 16)], idx_v)
            for jj in range(16):
                r = grow0 + jj
                cur, csem = ((rows_a, sem_a) if jj % 2 == 0
                             else (rows_b, sem_b))
                nxt, nsem = ((rows_b, sem_b) if jj % 2 == 0
                             else (rows_a, sem_a))
                if jj < 15:
                    pltpu.async_copy(_rows_src(r + 1), nxt, nsem)
                else:
                    @pl.when(r < bL)
                    def _():
                        pltpu.async_copy(_rows_src(r + 1), nxt, nsem)
                pltpu.make_async_copy(_rows_src(r), cur, csem).wait()
                pltpu.sync_copy(cur, acc.at[idx_v.at[jj]], add=True)
            return carry

        lax.fori_loop(0, _GPT // 2, _dgroup, 0)
        plsc.subcore_barrier()
        pltpu.sync_copy(acc.at[pl.ds(r0, _SLAB)],
                        out_ref.at[pl.ds(r0, _SLAB), pl.ds(ocol0, 128)])

    def _deg_pass(idx2d, out_ref):
        pltpu.sync_copy(zer, acc.at[pl.ds(r0, _SLAB)])
        plsc.subcore_barrier()

        def _group(g2, carry):
            grow0 = (s * _GPT + 2 * g2) * 8
            pltpu.sync_copy(idx2d.at[pl.ds(grow0, 16)], idx_v)
            for jj in range(16):
                pltpu.sync_copy(rows_a, acc.at[idx_v.at[jj]], add=True)
            return carry

        lax.fori_loop(0, _GPT // 2, _group, 0)
        plsc.subcore_barrier()
        pltpu.sync_copy(acc.at[pl.ds(r0, _SLAB)], out_ref.at[pl.ds(r0, _SLAB)])

    def _fill_ones(r, carry):
        for k2 in range(8):
            rows_a[r, pl.ds(16 * k2, 16)] = one16
        return carry

    lax.fori_loop(0, _NB, _fill_ones, 0)

    @pl.when(c == 0)
    def _():
        _deg_pass(dst2d, o_dinc)
        _pass(e1, 0, dst2d, o_inc1, 0)
        _pass(e1, 0, src2d, o_out1, 0)
        _pass(e1, 128, dst2d, o_inc1, 128)

    @pl.when(c == 1)
    def _():
        _deg_pass(src2d, o_dout)
        _pass(e1, 128, src2d, o_out1, 128)
        _pass(e2, 0, dst2d, o_inc2, 0)
        _pass(e2, 0, src2d, o_out2, 0)


@functools.lru_cache(maxsize=None)
def _make_sc_scatter():
    return pl.kernel(
        _sc_kernel_fn,
        mesh=plsc.VectorSubcoreMesh(core_axis_name="c", subcore_axis_name="s"),
        out_type=[
            jax.ShapeDtypeStruct((N_NP, 256), jnp.float32),
            jax.ShapeDtypeStruct((N_NP, 256), jnp.float32),
            jax.ShapeDtypeStruct((N_NP, 128), jnp.float32),
            jax.ShapeDtypeStruct((N_NP, 128), jnp.float32),
            jax.ShapeDtypeStruct((N_NP, 128), jnp.float32),
            jax.ShapeDtypeStruct((N_NP, 128), jnp.float32),
        ],
        scratch_types=[
            pltpu.VMEM((_NB, 128), jnp.float32),
            pltpu.VMEM((_NB, 128), jnp.float32),
            pltpu.VMEM((16, _NB), jnp.int32),
            pltpu.VMEM_SHARED((N_NP, 128), jnp.float32),
            pltpu.SemaphoreType.DMA,
            pltpu.SemaphoreType.DMA,
        ],
    )


def _sc_kernel_fn(e1, e2, dst2d, src2d, zer,
                  o_inc1, o_out1, o_inc2, o_out2, o_dinc, o_dout,
                  rows_a, rows_b, idx_v, acc, sem_a, sem_b):
    _sc_body(e1, e2, dst2d, src2d, zer,
             o_inc1, o_out1, o_inc2, o_out2, o_dinc, o_dout,
             rows_a, rows_b, idx_v, acc, sem_a, sem_b)


_NNB = 2000
_NNBLK = N_N // _NNB


def _k3_body(x, inc1, out1, inc2, out2, dinc, dout,
             wn1, win1, wout1, bn1, wn2, win2, wout2, bn2,
             se1, se2, u, a,
             wu1, wun1, wue1, bu1, wu2, wun2, wue2, bu2,
             wga, bga, wa1, ba1, wa2, ba2, wa3, ba3,
             q_o, sn1_v, sn2_v):
    i = pl.program_id(0)
    dot = lambda p, w: jnp.dot(p, w, preferred_element_type=jnp.float32)
    inv_di = 1.0 / jnp.maximum(dinc[...][:, 0:1], 1.0)
    inv_do = 1.0 / jnp.maximum(dout[...][:, 0:1], 1.0)

    n1 = dot(x[...], wn1[...])
    n1 += dot(inc1[...] * inv_di, win1[...])
    n1 += dot(out1[...] * inv_do, wout1[...])
    n1 = jax.nn.relu(n1 + bn1[...])

    n2 = dot(n1, wn2[...])
    n2 += dot(inc2[...] * inv_di, win2[...])
    n2 += dot(out2[...] * inv_do, wout2[...])
    n2 = jax.nn.relu(n2 + bn2[...])

    s1 = jnp.sum(n1, axis=0, keepdims=True)
    s2 = jnp.sum(n2, axis=0, keepdims=True)

    @pl.when(i == 0)
    def _():
        sn1_v[...] = s1
        sn2_v[...] = s2

    @pl.when(i > 0)
    def _():
        sn1_v[...] += s1
        sn2_v[...] += s2

    @pl.when(i == _NNBLK - 1)
    def _():
        mn1 = sn1_v[...] * (1.0 / N_N)
        mn2 = sn2_v[...] * (1.0 / N_N)
        me1 = se1[...] * (1.0 / N_E)
        me2 = se2[...] * (1.0 / N_E)
        u1 = jax.nn.relu(dot(u[...], wu1[...]) + dot(mn1, wun1[...])
                         + dot(me1, wue1[...]) + bu1[...])
        u2 = jax.nn.relu(dot(u1, wu2[...]) + dot(mn2, wun2[...])
                         + dot(me2, wue2[...]) + bu2[...])
        sv = dot(u2, wga[...]) + bga[...]
        h = jax.nn.relu(dot(sv, wa1[0:128, :]) + dot(a[...], wa1[128:160, :]) + ba1[...])
        h = jax.nn.relu(dot(h, wa2[...]) + ba2[...])
        q_o[...] = dot(h, wa3[...]) + ba3[...]


def _k3(x, inc1, out1, inc2, out2, dinc, dout, *weights):
    def whole(shape):
        return pl.BlockSpec(shape, lambda i, _n=len(shape): (0,) * _n)

    in_specs = [
        pl.BlockSpec((_NNB, 128), lambda i: (i, 0)),
        pl.BlockSpec((_NNB, 256), lambda i: (i, 0)),
        pl.BlockSpec((_NNB, 256), lambda i: (i, 0)),
        pl.BlockSpec((_NNB, 128), lambda i: (i, 0)),
        pl.BlockSpec((_NNB, 128), lambda i: (i, 0)),
        pl.BlockSpec((_NNB, 128), lambda i: (i, 0)),
        pl.BlockSpec((_NNB, 128), lambda i: (i, 0)),
    ] + [whole(w.shape) for w in weights]
    return pl.pallas_call(
        _k3_body,
        grid=(_NNBLK,),
        in_specs=in_specs,
        out_specs=pl.BlockSpec((1, 1), lambda i: (0, 0)),
        out_shape=jax.ShapeDtypeStruct((1, 1), jnp.float32),
        scratch_shapes=[pltpu.VMEM((1, 256), jnp.float32),
                        pltpu.VMEM((1, 128), jnp.float32)],
        compiler_params=pltpu.CompilerParams(
            dimension_semantics=("arbitrary",)),
    )(x, inc1, out1, inc2, out2, dinc, dout, *weights)


def kernel(x, edge_index, edge_attr, u, a,
           W_e1, b_e1,
           W_n1, W_in1, W_out1, b_n1,
           W_u1, W_un1, W_ue1, b_u1,
           W_e2, b_e2,
           W_n2, W_in2, W_out2, b_n2,
           W_u2, W_un2, W_ue2, b_u2,
           W_ga, b_ga,
           W_a1, b_a1, W_a2, b_a2, W_a3, b_a3):
    e1, e2, se1, se2 = _k1(edge_attr, W_e1, b_e1.reshape(1, -1),
                           W_e2, b_e2.reshape(1, -1))
    pad = jnp.full((_NROWP * _NB - N_E,), _PAD_IDX, jnp.int32)
    dst2d = jnp.concatenate([edge_index[1], pad]).reshape(_NROWP, _NB)
    src2d = jnp.concatenate([edge_index[0], pad]).reshape(_NROWP, _NB)
    zer = jnp.zeros((_SLAB, 128), jnp.float32)
    inc1, out1, inc2, out2, dinc, dout = _make_sc_scatter()(e1, e2, dst2d, src2d, zer)
    return _k3(x, inc1, out1, inc2, out2, dinc, dout,
               W_n1, W_in1, W_out1, b_n1.reshape(1, -1),
               W_n2, W_in2, W_out2, b_n2.reshape(1, -1),
               se1, se2, u, a,
               W_u1, W_un1, W_ue1, b_u1.reshape(1, -1),
               W_u2, W_un2, W_ue2, b_u2.reshape(1, -1),
               W_ga, b_ga.reshape(1, -1),
               W_a1, b_a1.reshape(1, -1), W_a2, b_a2.reshape(1, -1),
               W_a3, b_a3.reshape(1, -1))


# speedup vs baseline: 1.1171x; 1.0218x over previous
import functools
import jax
import jax.numpy as jnp
from jax import lax
from jax.experimental import pallas as pl
from jax.experimental.pallas import tpu as pltpu
from jax.experimental.pallas import tpu_sc as plsc

N_E = 320000
N_N = 10000
N_NP = 10240
_EB = 6400
_NBLK = N_E // _EB
_NB = 128
_NROW = N_E // _NB
_NROWP = 2560
_GPT = 20
_SLAB = N_NP // 16
_PAD_IDX = 10200


def _k1_body(ea, w1, b1, w2, b2, e1_o, e2_o, se1_o, se2_o):
    i = pl.program_id(0)
    e1 = jax.nn.relu(jnp.dot(ea[...], w1[...], preferred_element_type=jnp.float32) + b1[...])
    e2 = jax.nn.relu(jnp.dot(e1, w2[...], preferred_element_type=jnp.float32) + b2[...])
    e1_o[...] = e1
    e2_o[...] = e2
    s1 = jnp.sum(e1, axis=0, keepdims=True)
    s2 = jnp.sum(e2, axis=0, keepdims=True)

    @pl.when(i == 0)
    def _():
        se1_o[...] = s1
        se2_o[...] = s2

    @pl.when(i > 0)
    def _():
        se1_o[...] += s1
        se2_o[...] += s2


def _k1(ea, w1, b1, w2, b2):
    return pl.pallas_call(
        _k1_body,
        grid=(_NBLK,),
        in_specs=[
            pl.BlockSpec((_EB, 16), lambda i: (i, 0)),
            pl.BlockSpec((16, 256), lambda i: (0, 0)),
            pl.BlockSpec((1, 256), lambda i: (0, 0)),
            pl.BlockSpec((256, 128), lambda i: (0, 0)),
            pl.BlockSpec((1, 128), lambda i: (0, 0)),
        ],
        out_specs=[
            pl.BlockSpec((_EB, 256), lambda i: (i, 0)),
            pl.BlockSpec((_EB, 128), lambda i: (i, 0)),
            pl.BlockSpec((1, 256), lambda i: (0, 0)),
            pl.BlockSpec((1, 128), lambda i: (0, 0)),
        ],
        out_shape=[
            jax.ShapeDtypeStruct((N_E, 256), jnp.float32),
            jax.ShapeDtypeStruct((N_E, 128), jnp.float32),
            jax.ShapeDtypeStruct((1, 256), jnp.float32),
            jax.ShapeDtypeStruct((1, 128), jnp.float32),
        ],
        compiler_params=pltpu.CompilerParams(
            dimension_semantics=("arbitrary",)),
    )(ea, w1, b1, w2, b2)


def _sc_body(e1, e2, dst2d, src2d, zer,
             o_inc1, o_out1, o_inc2, o_out2, o_dinc, o_dout,
             rows_a, rows_b, idx_v, acc, sem_a, sem_b):
    c = lax.axis_index("c")
    s = lax.axis_index("s")
    r0 = s * _SLAB
    b0 = s * _GPT * 8
    bL = b0 + _GPT * 8 - 1
    one16 = jnp.ones((16,), jnp.float32)

    def _pass(mat_ref, col0, idx2d, out_ref, ocol0):
        pltpu.sync_copy(zer, acc.at[pl.ds(r0, _SLAB)])
        plsc.subcore_barrier()

        def _rows_src(r):
            base = jnp.minimum(r, _NROW - 1) * _NB
            return mat_ref.at[pl.ds(base, _NB), pl.ds(col0, 128)]

        pltpu.async_copy(_rows_src(b0), rows_a, sem_a)

        def _dgroup(g2, carry):
            grow0 = (s * _GPT + 2 * g2) * 8
            pltpu.sync_copy(idx2d.at[pl.ds(grow0, 16)], idx_v)
            for jj in range(16):
                r = grow0 + jj
                cur, csem = ((rows_a, sem_a) if jj % 2 == 0
                             else (rows_b, sem_b))
                nxt, nsem = ((rows_b, sem_b) if jj % 2 == 0
                             else (rows_a, sem_a))
                if jj < 15:
                    pltpu.async_copy(_rows_src(r + 1), nxt, nsem)
                else:
                    @pl.when(r < bL)
                    def _():
                        pltpu.async_copy(_rows_src(r + 1), nxt, nsem)
                pltpu.make_async_copy(_rows_src(r), cur, csem).wait()
                pltpu.sync_copy(cur, acc.at[idx_v.at[jj]], add=True)
            return carry

        lax.fori_loop(0, _GPT // 2, _dgroup, 0)
        plsc.subcore_barrier()
        pltpu.sync_copy(acc.at[pl.ds(r0, _SLAB)],
                        out_ref.at[pl.ds(r0, _SLAB), pl.ds(ocol0, 128)])

    def _deg_pass(idx2d, out_ref):
        pltpu.sync_copy(zer, acc.at[pl.ds(r0, _SLAB)])
        plsc.subcore_barrier()

        def _group(g2, carry):
            grow0 = (s * _GPT + 2 * g2) * 8
            pltpu.sync_copy(idx2d.at[pl.ds(grow0, 16)], idx_v)
            for jj in range(16):
                pltpu.sync_copy(rows_a, acc.at[idx_v.at[jj]], add=True)
            return carry

        lax.fori_loop(0, _GPT // 2, _group, 0)
        plsc.subcore_barrier()
        pltpu.sync_copy(acc.at[pl.ds(r0, _SLAB)], out_ref.at[pl.ds(r0, _SLAB)])

    def _fill_ones(r, carry):
        for k2 in range(8):
            rows_a[r, pl.ds(16 * k2, 16)] = one16
        return carry

    lax.fori_loop(0, _NB, _fill_ones, 0)

    @pl.when(c == 0)
    def _():
        _deg_pass(dst2d, o_dinc)
        _pass(e1, 0, dst2d, o_inc1, 0)
        _pass(e1, 0, src2d, o_out1, 0)
        _pass(e1, 128, dst2d, o_inc1, 128)

    @pl.when(c == 1)
    def _():
        _deg_pass(src2d, o_dout)
        _pass(e1, 128, src2d, o_out1, 128)
        _pass(e2, 0, dst2d, o_inc2, 0)
        _pass(e2, 0, src2d, o_out2, 0)


@functools.lru_cache(maxsize=None)
def _make_sc_scatter():
    return pl.kernel(
        _sc_kernel_fn,
        mesh=plsc.VectorSubcoreMesh(core_axis_name="c", subcore_axis_name="s"),
        out_type=[
            jax.ShapeDtypeStruct((N_NP, 256), jnp.float32),
            jax.ShapeDtypeStruct((N_NP, 256), jnp.float32),
            jax.ShapeDtypeStruct((N_NP, 128), jnp.float32),
            jax.ShapeDtypeStruct((N_NP, 128), jnp.float32),
            jax.ShapeDtypeStruct((N_NP, 128), jnp.float32),
            jax.ShapeDtypeStruct((N_NP, 128), jnp.float32),
        ],
        scratch_types=[
            pltpu.VMEM((_NB, 128), jnp.float32),
            pltpu.VMEM((_NB, 128), jnp.float32),
            pltpu.VMEM((16, _NB), jnp.int32),
            pltpu.VMEM_SHARED((N_NP, 128), jnp.float32),
            pltpu.SemaphoreType.DMA,
            pltpu.SemaphoreType.DMA,
        ],
    )


def _sc_kernel_fn(e1, e2, dst2d, src2d, zer,
                  o_inc1, o_out1, o_inc2, o_out2, o_dinc, o_dout,
                  rows_a, rows_b, idx_v, acc, sem_a, sem_b):
    _sc_body(e1, e2, dst2d, src2d, zer,
             o_inc1, o_out1, o_inc2, o_out2, o_dinc, o_dout,
             rows_a, rows_b, idx_v, acc, sem_a, sem_b)


_NNB = 2000
_NNBLK = N_N // _NNB


def _k3_body(x, inc1, out1, inc2, out2, dinc, dout,
             wn1, win1, wout1, bn1, wn2, win2, wout2, bn2,
             se1, se2, u, a,
             wu1, wun1, wue1, bu1, wu2, wun2, wue2, bu2,
             wga, bga, wa1, ba1, wa2, ba2, wa3, ba3,
             q_o, sn1_v, sn2_v):
    i = pl.program_id(0)
    dot = lambda p, w: jnp.dot(p, w, preferred_element_type=jnp.float32)
    inv_di = 1.0 / jnp.maximum(dinc[...][:, 0:1], 1.0)
    inv_do = 1.0 / jnp.maximum(dout[...][:, 0:1], 1.0)

    n1 = dot(x[...], wn1[...])
    n1 += dot(inc1[...] * inv_di, win1[...])
    n1 += dot(out1[...] * inv_do, wout1[...])
    n1 = jax.nn.relu(n1 + bn1[...])

    n2 = dot(n1, wn2[...])
    n2 += dot(inc2[...] * inv_di, win2[...])
    n2 += dot(out2[...] * inv_do, wout2[...])
    n2 = jax.nn.relu(n2 + bn2[...])

    s1 = jnp.sum(n1, axis=0, keepdims=True)
    s2 = jnp.sum(n2, axis=0, keepdims=True)

    @pl.when(i == 0)
    def _():
        sn1_v[...] = s1
        sn2_v[...] = s2

    @pl.when(i > 0)
    def _():
        sn1_v[...] += s1
        sn2_v[...] += s2

    @pl.when(i == _NNBLK - 1)
    def _():
        mn1 = sn1_v[...] * (1.0 / N_N)
        mn2 = sn2_v[...] * (1.0 / N_N)
        me1 = se1[...] * (1.0 / N_E)
        me2 = se2[...] * (1.0 / N_E)
        u1 = jax.nn.relu(dot(u[...], wu1[...]) + dot(mn1, wun1[...])
                         + dot(me1, wue1[...]) + bu1[...])
        u2 = jax.nn.relu(dot(u1, wu2[...]) + dot(mn2, wun2[...])
                         + dot(me2, wue2[...]) + bu2[...])
        sv = dot(u2, wga[...]) + bga[...]
        h = jax.nn.relu(dot(sv, wa1[0:128, :]) + dot(a[...], wa1[128:160, :]) + ba1[...])
        h = jax.nn.relu(dot(h, wa2[...]) + ba2[...])
        q_o[...] = dot(h, wa3[...]) + ba3[...]


def _k3(x, inc1, out1, inc2, out2, dinc, dout, *weights):
    def whole(shape):
        return pl.BlockSpec(shape, lambda i, _n=len(shape): (0,) * _n)

    in_specs = [
        pl.BlockSpec((_NNB, 128), lambda i: (i, 0)),
        pl.BlockSpec((_NNB, 256), lambda i: (i, 0)),
        pl.BlockSpec((_NNB, 256), lambda i: (i, 0)),
        pl.BlockSpec((_NNB, 128), lambda i: (i, 0)),
        pl.BlockSpec((_NNB, 128), lambda i: (i, 0)),
        pl.BlockSpec((_NNB, 128), lambda i: (i, 0)),
        pl.BlockSpec((_NNB, 128), lambda i: (i, 0)),
    ] + [whole(w.shape) for w in weights]
    return pl.pallas_call(
        _k3_body,
        grid=(_NNBLK,),
        in_specs=in_specs,
        out_specs=pl.BlockSpec((1, 1), lambda i: (0, 0)),
        out_shape=jax.ShapeDtypeStruct((1, 1), jnp.float32),
        scratch_shapes=[pltpu.VMEM((1, 256), jnp.float32),
                        pltpu.VMEM((1, 128), jnp.float32)],
        compiler_params=pltpu.CompilerParams(
            dimension_semantics=("arbitrary",)),
    )(x, inc1, out1, inc2, out2, dinc, dout, *weights)


def kernel(x, edge_index, edge_attr, u, a,
           W_e1, b_e1,
           W_n1, W_in1, W_out1, b_n1,
           W_u1, W_un1, W_ue1, b_u1,
           W_e2, b_e2,
           W_n2, W_in2, W_out2, b_n2,
           W_u2, W_un2, W_ue2, b_u2,
           W_ga, b_ga,
           W_a1, b_a1, W_a2, b_a2, W_a3, b_a3):
    e1, e2, se1, se2 = _k1(edge_attr, W_e1, b_e1.reshape(1, -1),
                           W_e2, b_e2.reshape(1, -1))
    pad = jnp.full((_NROWP * _NB - N_E,), _PAD_IDX, jnp.int32)
    dst2d = jnp.concatenate([edge_index[1], pad]).reshape(_NROWP, _NB)
    src2d = jnp.concatenate([edge_index[0], pad]).reshape(_NROWP, _NB)
    zer = jnp.zeros((_SLAB, 128), jnp.float32)
    inc1, out1, inc2, out2, dinc, dout = _make_sc_scatter()(e1, e2, dst2d, src2d, zer)
    return _k3(x, inc1, out1, inc2, out2, dinc, dout,
               W_n1, W_in1, W_out1, b_n1.reshape(1, -1),
               W_n2, W_in2, W_out2, b_n2.reshape(1, -1),
               se1, se2, u, a,
               W_u1, W_un1, W_ue1, b_u1.reshape(1, -1),
               W_u2, W_un2, W_ue2, b_u2.reshape(1, -1),
               W_ga, b_ga.reshape(1, -1),
               W_a1, b_a1.reshape(1, -1), W_a2, b_a2.reshape(1, -1),
               W_a3, b_a3.reshape(1, -1))


# speedup vs baseline: 1.1207x; 1.0032x over previous
import functools
import jax
import jax.numpy as jnp
from jax import lax
from jax.experimental import pallas as pl
from jax.experimental.pallas import tpu as pltpu
from jax.experimental.pallas import tpu_sc as plsc

N_E = 320000
N_N = 10000
N_NP = 10240
_EB = 12800
_NBLK = N_E // _EB
_NB = 128
_NROW = N_E // _NB
_NROWP = 2560
_GPT = 20
_SLAB = N_NP // 16
_PAD_IDX = 10200


def _k1_body(ea, w1, b1, w2, b2, e1_o, e2_o, se1_o, se2_o):
    i = pl.program_id(0)
    e1 = jax.nn.relu(jnp.dot(ea[...], w1[...], preferred_element_type=jnp.float32) + b1[...])
    e2 = jax.nn.relu(jnp.dot(e1, w2[...], preferred_element_type=jnp.float32) + b2[...])
    e1_o[...] = e1
    e2_o[...] = e2
    s1 = jnp.sum(e1, axis=0, keepdims=True)
    s2 = jnp.sum(e2, axis=0, keepdims=True)

    @pl.when(i == 0)
    def _():
        se1_o[...] = s1
        se2_o[...] = s2

    @pl.when(i > 0)
    def _():
        se1_o[...] += s1
        se2_o[...] += s2


def _k1(ea, w1, b1, w2, b2):
    return pl.pallas_call(
        _k1_body,
        grid=(_NBLK,),
        in_specs=[
            pl.BlockSpec((_EB, 16), lambda i: (i, 0)),
            pl.BlockSpec((16, 256), lambda i: (0, 0)),
            pl.BlockSpec((1, 256), lambda i: (0, 0)),
            pl.BlockSpec((256, 128), lambda i: (0, 0)),
            pl.BlockSpec((1, 128), lambda i: (0, 0)),
        ],
        out_specs=[
            pl.BlockSpec((_EB, 256), lambda i: (i, 0)),
            pl.BlockSpec((_EB, 128), lambda i: (i, 0)),
            pl.BlockSpec((1, 256), lambda i: (0, 0)),
            pl.BlockSpec((1, 128), lambda i: (0, 0)),
        ],
        out_shape=[
            jax.ShapeDtypeStruct((N_E, 256), jnp.float32),
            jax.ShapeDtypeStruct((N_E, 128), jnp.float32),
            jax.ShapeDtypeStruct((1, 256), jnp.float32),
            jax.ShapeDtypeStruct((1, 128), jnp.float32),
        ],
        compiler_params=pltpu.CompilerParams(
            dimension_semantics=("arbitrary",)),
    )(ea, w1, b1, w2, b2)


def _sc_body(e1, e2, dst2d, src2d, zer,
             o_inc1, o_out1, o_inc2, o_out2, o_dinc, o_dout,
             rows_a, rows_b, idx_v, acc, sem_a, sem_b):
    c = lax.axis_index("c")
    s = lax.axis_index("s")
    r0 = s * _SLAB
    b0 = s * _GPT * 8
    bL = b0 + _GPT * 8 - 1
    one16 = jnp.ones((16,), jnp.float32)

    def _pass(mat_ref, col0, idx2d, out_ref, ocol0):
        pltpu.sync_copy(zer, acc.at[pl.ds(r0, _SLAB)])
        plsc.subcore_barrier()

        def _rows_src(r):
            base = jnp.minimum(r, _NROW - 1) * _NB
            return mat_ref.at[pl.ds(base, _NB), pl.ds(col0, 128)]

        pltpu.async_copy(_rows_src(b0), rows_a, sem_a)

        def _dgroup(g2, carry):
            grow0 = (s * _GPT + 2 * g2) * 8
            pltpu.sync_copy(idx2d.at[pl.ds(grow0, 16)], idx_v)
            for jj in range(16):
                r = grow0 + jj
                cur, csem = ((rows_a, sem_a) if jj % 2 == 0
                             else (rows_b, sem_b))
                nxt, nsem = ((rows_b, sem_b) if jj % 2 == 0
                             else (rows_a, sem_a))
                if jj < 15:
                    pltpu.async_copy(_rows_src(r + 1), nxt, nsem)
                else:
                    @pl.when(r < bL)
                    def _():
                        pltpu.async_copy(_rows_src(r + 1), nxt, nsem)
                pltpu.make_async_copy(_rows_src(r), cur, csem).wait()
                pltpu.sync_copy(cur, acc.at[idx_v.at[jj]], add=True)
            return carry

        lax.fori_loop(0, _GPT // 2, _dgroup, 0)
        plsc.subcore_barrier()
        pltpu.sync_copy(acc.at[pl.ds(r0, _SLAB)],
                        out_ref.at[pl.ds(r0, _SLAB), pl.ds(ocol0, 128)])

    def _deg_pass(idx2d, out_ref):
        pltpu.sync_copy(zer, acc.at[pl.ds(r0, _SLAB)])
        plsc.subcore_barrier()

        def _group(g2, carry):
            grow0 = (s * _GPT + 2 * g2) * 8
            pltpu.sync_copy(idx2d.at[pl.ds(grow0, 16)], idx_v)
            for jj in range(16):
                pltpu.sync_copy(rows_a, acc.at[idx_v.at[jj]], add=True)
            return carry

        lax.fori_loop(0, _GPT // 2, _group, 0)
        plsc.subcore_barrier()
        pltpu.sync_copy(acc.at[pl.ds(r0, _SLAB)], out_ref.at[pl.ds(r0, _SLAB)])

    def _fill_ones(r, carry):
        for k2 in range(8):
            rows_a[r, pl.ds(16 * k2, 16)] = one16
        return carry

    lax.fori_loop(0, _NB, _fill_ones, 0)

    @pl.when(c == 0)
    def _():
        _deg_pass(dst2d, o_dinc)
        _pass(e1, 0, dst2d, o_inc1, 0)
        _pass(e1, 0, src2d, o_out1, 0)
        _pass(e1, 128, dst2d, o_inc1, 128)

    @pl.when(c == 1)
    def _():
        _deg_pass(src2d, o_dout)
        _pass(e1, 128, src2d, o_out1, 128)
        _pass(e2, 0, dst2d, o_inc2, 0)
        _pass(e2, 0, src2d, o_out2, 0)


@functools.lru_cache(maxsize=None)
def _make_sc_scatter():
    return pl.kernel(
        _sc_kernel_fn,
        mesh=plsc.VectorSubcoreMesh(core_axis_name="c", subcore_axis_name="s"),
        out_type=[
            jax.ShapeDtypeStruct((N_NP, 256), jnp.float32),
            jax.ShapeDtypeStruct((N_NP, 256), jnp.float32),
            jax.ShapeDtypeStruct((N_NP, 128), jnp.float32),
            jax.ShapeDtypeStruct((N_NP, 128), jnp.float32),
            jax.ShapeDtypeStruct((N_NP, 128), jnp.float32),
            jax.ShapeDtypeStruct((N_NP, 128), jnp.float32),
        ],
        scratch_types=[
            pltpu.VMEM((_NB, 128), jnp.float32),
            pltpu.VMEM((_NB, 128), jnp.float32),
            pltpu.VMEM((16, _NB), jnp.int32),
            pltpu.VMEM_SHARED((N_NP, 128), jnp.float32),
            pltpu.SemaphoreType.DMA,
            pltpu.SemaphoreType.DMA,
        ],
    )


def _sc_kernel_fn(e1, e2, dst2d, src2d, zer,
                  o_inc1, o_out1, o_inc2, o_out2, o_dinc, o_dout,
                  rows_a, rows_b, idx_v, acc, sem_a, sem_b):
    _sc_body(e1, e2, dst2d, src2d, zer,
             o_inc1, o_out1, o_inc2, o_out2, o_dinc, o_dout,
             rows_a, rows_b, idx_v, acc, sem_a, sem_b)


_NNB = 2000
_NNBLK = N_N // _NNB


def _k3_body(x, inc1, out1, inc2, out2, dinc, dout,
             wn1, win1, wout1, bn1, wn2, win2, wout2, bn2,
             se1, se2, u, a,
             wu1, wun1, wue1, bu1, wu2, wun2, wue2, bu2,
             wga, bga, wa1, ba1, wa2, ba2, wa3, ba3,
             q_o, sn1_v, sn2_v):
    i = pl.program_id(0)
    dot = lambda p, w: jnp.dot(p, w, preferred_element_type=jnp.float32)
    inv_di = 1.0 / jnp.maximum(dinc[...][:, 0:1], 1.0)
    inv_do = 1.0 / jnp.maximum(dout[...][:, 0:1], 1.0)

    n1 = dot(x[...], wn1[...])
    n1 += dot(inc1[...] * inv_di, win1[...])
    n1 += dot(out1[...] * inv_do, wout1[...])
    n1 = jax.nn.relu(n1 + bn1[...])

    n2 = dot(n1, wn2[...])
    n2 += dot(inc2[...] * inv_di, win2[...])
    n2 += dot(out2[...] * inv_do, wout2[...])
    n2 = jax.nn.relu(n2 + bn2[...])

    s1 = jnp.sum(n1, axis=0, keepdims=True)
    s2 = jnp.sum(n2, axis=0, keepdims=True)

    @pl.when(i == 0)
    def _():
        sn1_v[...] = s1
        sn2_v[...] = s2

    @pl.when(i > 0)
    def _():
        sn1_v[...] += s1
        sn2_v[...] += s2

    @pl.when(i == _NNBLK - 1)
    def _():
        mn1 = sn1_v[...] * (1.0 / N_N)
        mn2 = sn2_v[...] * (1.0 / N_N)
        me1 = se1[...] * (1.0 / N_E)
        me2 = se2[...] * (1.0 / N_E)
        u1 = jax.nn.relu(dot(u[...], wu1[...]) + dot(mn1, wun1[...])
                         + dot(me1, wue1[...]) + bu1[...])
        u2 = jax.nn.relu(dot(u1, wu2[...]) + dot(mn2, wun2[...])
                         + dot(me2, wue2[...]) + bu2[...])
        sv = dot(u2, wga[...]) + bga[...]
        h = jax.nn.relu(dot(sv, wa1[0:128, :]) + dot(a[...], wa1[128:160, :]) + ba1[...])
        h = jax.nn.relu(dot(h, wa2[...]) + ba2[...])
        q_o[...] = dot(h, wa3[...]) + ba3[...]


def _k3(x, inc1, out1, inc2, out2, dinc, dout, *weights):
    def whole(shape):
        return pl.BlockSpec(shape, lambda i, _n=len(shape): (0,) * _n)

    in_specs = [
        pl.BlockSpec((_NNB, 128), lambda i: (i, 0)),
        pl.BlockSpec((_NNB, 256), lambda i: (i, 0)),
        pl.BlockSpec((_NNB, 256), lambda i: (i, 0)),
        pl.BlockSpec((_NNB, 128), lambda i: (i, 0)),
        pl.BlockSpec((_NNB, 128), lambda i: (i, 0)),
        pl.BlockSpec((_NNB, 128), lambda i: (i, 0)),
        pl.BlockSpec((_NNB, 128), lambda i: (i, 0)),
    ] + [whole(w.shape) for w in weights]
    return pl.pallas_call(
        _k3_body,
        grid=(_NNBLK,),
        in_specs=in_specs,
        out_specs=pl.BlockSpec((1, 1), lambda i: (0, 0)),
        out_shape=jax.ShapeDtypeStruct((1, 1), jnp.float32),
        scratch_shapes=[pltpu.VMEM((1, 256), jnp.float32),
                        pltpu.VMEM((1, 128), jnp.float32)],
        compiler_params=pltpu.CompilerParams(
            dimension_semantics=("arbitrary",)),
    )(x, inc1, out1, inc2, out2, dinc, dout, *weights)


def kernel(x, edge_index, edge_attr, u, a,
           W_e1, b_e1,
           W_n1, W_in1, W_out1, b_n1,
           W_u1, W_un1, W_ue1, b_u1,
           W_e2, b_e2,
           W_n2, W_in2, W_out2, b_n2,
           W_u2, W_un2, W_ue2, b_u2,
           W_ga, b_ga,
           W_a1, b_a1, W_a2, b_a2, W_a3, b_a3):
    e1, e2, se1, se2 = _k1(edge_attr, W_e1, b_e1.reshape(1, -1),
                           W_e2, b_e2.reshape(1, -1))
    pad = jnp.full((_NROWP * _NB - N_E,), _PAD_IDX, jnp.int32)
    dst2d = jnp.concatenate([edge_index[1], pad]).reshape(_NROWP, _NB)
    src2d = jnp.concatenate([edge_index[0], pad]).reshape(_NROWP, _NB)
    zer = jnp.zeros((_SLAB, 128), jnp.float32)
    inc1, out1, inc2, out2, dinc, dout = _make_sc_scatter()(e1, e2, dst2d, src2d, zer)
    return _k3(x, inc1, out1, inc2, out2, dinc, dout,
               W_n1, W_in1, W_out1, b_n1.reshape(1, -1),
               W_n2, W_in2, W_out2, b_n2.reshape(1, -1),
               se1, se2, u, a,
               W_u1, W_un1, W_ue1, b_u1.reshape(1, -1),
               W_u2, W_un2, W_ue2, b_u2.reshape(1, -1),
               W_ga, b_ga.reshape(1, -1),
               W_a1, b_a1.reshape(1, -1), W_a2, b_a2.reshape(1, -1),
               W_a3, b_a3.reshape(1, -1))
